# baseline (speedup 1.0000x reference)
import functools

import jax
import jax.numpy as jnp
from jax import lax
from jax.experimental import pallas as pl
from jax.experimental.pallas import tpu as pltpu
from jax.experimental.pallas import tpu_sc as plsc

N = 10000
D = 128
E = 320000
NC = 2
NS = 16
NW = NC * NS
EPW = E // NW

_SC_MESH = plsc.VectorSubcoreMesh(
    core_axis_name="c", subcore_axis_name="s", num_cores=NC, num_subcores=NS)


_DEG_CHUNK = 2000


def _degrees_body(e0s, e0d, e1s, e1d, e2s, e2d, deg_hbm,
                  idx_v, ones_v, zeros_v, d0, d1, d2, d3, d4, d5):
  c = lax.axis_index("c")
  s = lax.axis_index("s")
  degs = [d0, d1, d2, d3, d4, d5]

  def init_ones(i, _):
    ones_v[pl.ds(i * 16, 16)] = jnp.full((16,), 1.0, jnp.float32)
    return 0
  lax.fori_loop(0, _DEG_CHUNK // 16, init_ones, 0)

  def init_zeros(i, _):
    zeros_v[pl.ds(i * 16, 16)] = jnp.full((16,), 0.0, jnp.float32)
    return 0
  lax.fori_loop(0, N // 16, init_zeros, 0)

  for j in range(6):
    @pl.when(s == j)
    def _():
      pltpu.sync_copy(zeros_v, degs[j])
  plsc.subcore_barrier()

  w = c * NS + s
  base_w = w * EPW
  for j, e in enumerate((e0s, e0d, e1s, e1d, e2s, e2d)):
    dref = degs[j]

    def chunk(i, _):
      base = base_w + i * _DEG_CHUNK
      pltpu.sync_copy(e.at[pl.ds(base, _DEG_CHUNK)], idx_v)
      pltpu.sync_copy(ones_v, dref.at[idx_v], add=True)
      return 0
    lax.fori_loop(0, EPW // _DEG_CHUNK, chunk, 0)

  plsc.subcore_barrier()
  for j in range(6):
    @pl.when(s == j)
    def _():
      pltpu.sync_copy(degs[j], zeros_v)
      pltpu.sync_copy(zeros_v, deg_hbm.at[pl.ds((c * 6 + j) * N, N)])


_degrees = pl.kernel(
    _degrees_body,
    out_type=jax.ShapeDtypeStruct((NC * 6 * N,), jnp.float32),
    mesh=_SC_MESH,
    compiler_params=pltpu.CompilerParams(needs_layout_passes=False),
    scratch_types=[
        pltpu.VMEM((_DEG_CHUNK,), jnp.int32),
        pltpu.VMEM((_DEG_CHUNK,), jnp.float32),
        pltpu.VMEM((N,), jnp.float32),
    ] + [pltpu.VMEM_SHARED((N,), jnp.float32) for _ in range(6)],
)


def _norms_body(deg_ref, norm_ref):
  deg = deg_ref[0] + deg_ref[1]
  norm_ref[...] = jnp.where(
      deg > 0.0, lax.rsqrt(jnp.maximum(deg, 1e-12)), 0.0)


_norms = pl.pallas_call(
    _norms_body,
    out_shape=jax.ShapeDtypeStruct((6, N), jnp.float32),
)

_PROJ_BLK = 2000


def _project_body(x_ref, wa_ref, wb_ref, wc_ref, ya_ref, yb_ref, yc_ref):
  x = x_ref[...]
  ya_ref[...] = jnp.dot(x, wa_ref[...], preferred_element_type=jnp.float32)
  yb_ref[...] = jnp.dot(x, wb_ref[...], preferred_element_type=jnp.float32)
  yc_ref[...] = jnp.dot(x, wc_ref[...], preferred_element_type=jnp.float32)


_project = pl.pallas_call(
    _project_body,
    grid=(N // _PROJ_BLK,),
    in_specs=[
        pl.BlockSpec((_PROJ_BLK, D), lambda i: (i, 0)),
        pl.BlockSpec((D, D), lambda i: (0, 0)),
        pl.BlockSpec((D, D), lambda i: (0, 0)),
        pl.BlockSpec((D, D), lambda i: (0, 0)),
    ],
    out_specs=[pl.BlockSpec((_PROJ_BLK, D), lambda i: (i, 0))] * 3,
    out_shape=[jax.ShapeDtypeStruct((N, D), jnp.float32)] * 3,
)


_CHUNK = 80
_BLK = 2000
_NCHUNK_B = _BLK // _CHUNK
_NBLK = EPW // _BLK
_RPT = 632
_RPT_LAST = N - 15 * _RPT


def _row_chunks(total):
  out, off = [], 0
  while off < total:
    n = min(_CHUNK, total - off)
    out.append((off, n))
    off += n
  return out


def _msgpass_body(y0, y1, y2, e0s, e0d, e1s, e1d, e2s, e2d, w0, w1, w2,
                  normsf, out_hbm, a_hbm,
                  nsrc_v, ndst_v, src_f, dst_f, ew_v, c_v, rows0, rows1,
                  sem0, sem1, sem2, sem3,
                  acc_sh, a0_sh, a1_sh, a2_sh):
  c = lax.axis_index("c")
  s = lax.axis_index("s")
  a_shs = [a0_sh, a1_sh, a2_sh]

  def init_zrows(i, _):
    for j in range(D // 16):
      rows0[i, pl.ds(j * 16, 16)] = jnp.full((16,), 0.0, jnp.float32)
    return 0
  lax.fori_loop(0, _CHUNK, init_zrows, 0)

  def init_znsrc(i, _):
    nsrc_v[pl.ds(i * 16, 16)] = jnp.full((16,), 0.0, jnp.float32)
    return 0
  lax.fori_loop(0, N // 16, init_znsrc, 0)

  start = s * _RPT
  @pl.when(s < 15)
  def _():
    for off, n in _row_chunks(_RPT):
      pltpu.sync_copy(rows0.at[pl.ds(0, n), :],
                      acc_sh.at[pl.ds(start + off, n), :])
  @pl.when(s == 15)
  def _():
    for off, n in _row_chunks(_RPT_LAST):
      pltpu.sync_copy(rows0.at[pl.ds(0, n), :],
                      acc_sh.at[pl.ds(15 * _RPT + off, n), :])
  for r in range(3):
    @pl.when(s == r)
    def _():
      pltpu.sync_copy(nsrc_v, a_shs[r])
  plsc.subcore_barrier()

  w = c * NS + s
  base_w = w * EPW

  def _scale(rows_ref, kc):
    @plsc.parallel_loop(0, _CHUNK, 1, unroll=4)
    def _(e):
      cb = plsc.load_gather(
          c_v, [jnp.full((16,), 0, jnp.int32) + (kc * _CHUNK + e)])
      for j in range(D // 16):
        rows_ref[e, pl.ds(j * 16, 16)] = rows_ref[e, pl.ds(j * 16, 16)] * cb

  for r, (y, es, ed, ew) in enumerate(
      ((y0, e0s, e0d, w0), (y1, e1s, e1d, w1), (y2, e2s, e2d, w2))):
    pltpu.sync_copy(normsf.at[pl.ds((2 * r) * N, N)], nsrc_v)
    pltpu.sync_copy(normsf.at[pl.ds((2 * r + 1) * N, N)], ndst_v)

    def block(bi, _):
      base = base_w + bi * _BLK
      ld0 = pltpu.async_copy(es.at[pl.ds(base, _BLK)], src_f, sem2)
      ld1 = pltpu.async_copy(ed.at[pl.ds(base, _BLK)], dst_f, sem2)
      ld2 = pltpu.async_copy(ew.at[pl.ds(base, _BLK)], ew_v, sem2)
      ld0.wait(); ld1.wait(); ld2.wait()

      def cgrp(g, _):
        s16 = src_f[pl.ds(g * 16, 16)]
        d16 = dst_f[pl.ds(g * 16, 16)]
        ns16 = plsc.load_gather(nsrc_v, [s16])
        nd16 = plsc.load_gather(ndst_v, [d16])
        c_v[pl.ds(g * 16, 16)] = ns16 * nd16 * ew_v[pl.ds(g * 16, 16)]
        return 0
      lax.fori_loop(0, _BLK // 16, cgrp, 0)

      a_scat = pltpu.async_copy(c_v, a_shs[r].at[src_f], sem3, add=True)

      pltpu.async_copy(y.at[src_f.at[pl.ds(0, _CHUNK)]], rows0, sem0)

      def two(k2, _):
        k = 2 * k2
        pltpu.async_copy(
            y.at[src_f.at[pl.ds((k + 1) * _CHUNK, _CHUNK)]], rows1, sem1)
        pltpu.make_async_copy(
            y.at[src_f.at[pl.ds(0, _CHUNK)]], rows0, sem0).wait()
        _scale(rows0, k)
        sc0 = pltpu.async_copy(
            rows0, acc_sh.at[dst_f.at[pl.ds(k * _CHUNK, _CHUNK)]], sem2,
            add=True)
        pltpu.make_async_copy(
            y.at[src_f.at[pl.ds(0, _CHUNK)]], rows1, sem1).wait()
        _scale(rows1, k + 1)
        sc1 = pltpu.async_copy(
            rows1, acc_sh.at[dst_f.at[pl.ds((k + 1) * _CHUNK, _CHUNK)]], sem3,
            add=True)
        sc0.wait()
        pltpu.async_copy(
            y.at[src_f.at[pl.ds((k + 2) * _CHUNK, _CHUNK)]], rows0, sem0)
        sc1.wait()
        return 0
      lax.fori_loop(0, (_NCHUNK_B - 1) // 2, two, 0)

      last = _NCHUNK_B - 1
      pltpu.make_async_copy(
          y.at[src_f.at[pl.ds(0, _CHUNK)]], rows0, sem0).wait()
      _scale(rows0, last)
      pltpu.sync_copy(rows0, acc_sh.at[dst_f.at[pl.ds(last * _CHUNK, _CHUNK)]],
                      add=True)
      a_scat.wait()
      return 0
    lax.fori_loop(0, _NBLK, block, 0)

  plsc.subcore_barrier()

  def _flush(row0, nrows):
    pltpu.sync_copy(acc_sh.at[pl.ds(row0, nrows), :],
                    rows0.at[pl.ds(0, nrows), :])
    pltpu.sync_copy(rows0.at[pl.ds(0, nrows), :],
                    out_hbm.at[pl.ds(c * N + row0, nrows), :])

  @pl.when(s < 15)
  def _():
    for off, n in _row_chunks(_RPT):
      _flush(start + off, n)
  @pl.when(s == 15)
  def _():
    for off, n in _row_chunks(_RPT_LAST):
      _flush(15 * _RPT + off, n)
  for r in range(3):
    @pl.when(s == r)
    def _():
      pltpu.sync_copy(a_shs[r], nsrc_v)
      pltpu.sync_copy(nsrc_v, a_hbm.at[pl.ds((c * 3 + r) * N, N)])


_msgpass = pl.kernel(
    _msgpass_body,
    out_type=[
        jax.ShapeDtypeStruct((NC * N, D), jnp.float32),
        jax.ShapeDtypeStruct((NC * 3 * N,), jnp.float32),
    ],
    mesh=_SC_MESH,
    compiler_params=pltpu.CompilerParams(needs_layout_passes=False),
    scratch_types=[
        pltpu.VMEM((N,), jnp.float32),
        pltpu.VMEM((N,), jnp.float32),
        pltpu.VMEM((_BLK,), jnp.int32),
        pltpu.VMEM((_BLK,), jnp.int32),
        pltpu.VMEM((_BLK,), jnp.float32),
        pltpu.VMEM((_BLK,), jnp.float32),
        pltpu.VMEM((_CHUNK, D), jnp.float32),
        pltpu.VMEM((_CHUNK, D), jnp.float32),
        pltpu.SemaphoreType.DMA,
        pltpu.SemaphoreType.DMA,
        pltpu.SemaphoreType.DMA,
        pltpu.SemaphoreType.DMA,
        pltpu.VMEM_SHARED((N, D), jnp.float32),
        pltpu.VMEM_SHARED((N,), jnp.float32),
        pltpu.VMEM_SHARED((N,), jnp.float32),
        pltpu.VMEM_SHARED((N,), jnp.float32),
    ],
)


def _final_body(out1_ref, a_ref, wa_ref, wb_ref, wc_ref, hg_ref):
  h1 = jax.nn.relu(out1_ref[0] + out1_ref[1])
  a = a_ref[0] + a_ref[1]
  sums = jnp.dot(a, h1, preferred_element_type=jnp.float32)
  hg = (jnp.dot(sums[0:1], wa_ref[...], preferred_element_type=jnp.float32)
        + jnp.dot(sums[1:2], wb_ref[...], preferred_element_type=jnp.float32)
        + jnp.dot(sums[2:3], wc_ref[...], preferred_element_type=jnp.float32))
  hg_ref[...] = hg * (1.0 / N)


_final = pl.pallas_call(
    _final_body,
    out_shape=jax.ShapeDtypeStruct((1, D), jnp.float32),
)


@jax.jit
def kernel(x, edge_index_r0, edge_index_r1, edge_index_r2,
           w_r0, w_r1, w_r2,
           W0_r0, W0_r1, W0_r2, W1_r0, W1_r1, W1_r2):
  es = [edge_index_r0.astype(jnp.int32), edge_index_r1.astype(jnp.int32),
        edge_index_r2.astype(jnp.int32)]
  e0s, e0d = es[0][0], es[0][1]
  e1s, e1d = es[1][0], es[1][1]
  e2s, e2d = es[2][0], es[2][1]
  degf = _degrees(e0s, e0d, e1s, e1d, e2s, e2d)
  norms = _norms(degf.reshape(NC, 6, N))
  y0, y1, y2 = _project(x, W0_r0, W0_r1, W0_r2)
  out1f, af = _msgpass(y0, y1, y2, e0s, e0d, e1s, e1d, e2s, e2d,
                       w_r0, w_r1, w_r2, norms.reshape(-1))
  return _final(out1f.reshape(NC, N, D), af.reshape(NC, 3, N),
                W1_r0, W1_r1, W1_r2)


# speedup vs baseline: 1.0917x; 1.0917x over previous
import functools

import jax
import jax.numpy as jnp
from jax import lax
from jax.experimental import pallas as pl
from jax.experimental.pallas import tpu as pltpu
from jax.experimental.pallas import tpu_sc as plsc

N = 10000
D = 128
E = 320000
NC = 2
NS = 16
NW = NC * NS
EPW = E // NW

_SC_MESH = plsc.VectorSubcoreMesh(
    core_axis_name="c", subcore_axis_name="s", num_cores=NC, num_subcores=NS)


_DEG_CHUNK = 2000


def _degrees_body(e0s, e0d, e1s, e1d, e2s, e2d, deg_hbm,
                  idx_v, ones_v, zeros_v, d0, d1, d2, d3, d4, d5):
  c = lax.axis_index("c")
  s = lax.axis_index("s")
  degs = [d0, d1, d2, d3, d4, d5]

  def init_ones(i, _):
    ones_v[pl.ds(i * 16, 16)] = jnp.full((16,), 1.0, jnp.float32)
    return 0
  lax.fori_loop(0, _DEG_CHUNK // 16, init_ones, 0)

  def init_zeros(i, _):
    zeros_v[pl.ds(i * 16, 16)] = jnp.full((16,), 0.0, jnp.float32)
    return 0
  lax.fori_loop(0, N // 16, init_zeros, 0)

  for j in range(6):
    @pl.when(s == j)
    def _():
      pltpu.sync_copy(zeros_v, degs[j])
  plsc.subcore_barrier()

  w = c * NS + s
  base_w = w * EPW
  for j, e in enumerate((e0s, e0d, e1s, e1d, e2s, e2d)):
    dref = degs[j]

    def chunk(i, _):
      base = base_w + i * _DEG_CHUNK
      pltpu.sync_copy(e.at[pl.ds(base, _DEG_CHUNK)], idx_v)
      pltpu.sync_copy(ones_v, dref.at[idx_v], add=True)
      return 0
    lax.fori_loop(0, EPW // _DEG_CHUNK, chunk, 0)

  plsc.subcore_barrier()
  for j in range(6):
    @pl.when(s == j)
    def _():
      pltpu.sync_copy(degs[j], zeros_v)
      pltpu.sync_copy(zeros_v, deg_hbm.at[pl.ds((c * 6 + j) * N, N)])


_degrees = pl.kernel(
    _degrees_body,
    out_type=jax.ShapeDtypeStruct((NC * 6 * N,), jnp.float32),
    mesh=_SC_MESH,
    compiler_params=pltpu.CompilerParams(needs_layout_passes=False),
    scratch_types=[
        pltpu.VMEM((_DEG_CHUNK,), jnp.int32),
        pltpu.VMEM((_DEG_CHUNK,), jnp.float32),
        pltpu.VMEM((N,), jnp.float32),
    ] + [pltpu.VMEM_SHARED((N,), jnp.float32) for _ in range(6)],
)


def _norms_body(deg_ref, norm_ref):
  deg = deg_ref[0] + deg_ref[1]
  norm_ref[...] = jnp.where(
      deg > 0.0, lax.rsqrt(jnp.maximum(deg, 1e-12)), 0.0)


_norms = pl.pallas_call(
    _norms_body,
    out_shape=jax.ShapeDtypeStruct((6, N), jnp.float32),
)

_PROJ_BLK = 2000


def _project_body(x_ref, wa_ref, wb_ref, wc_ref, ya_ref, yb_ref, yc_ref):
  x = x_ref[...]
  ya_ref[...] = jnp.dot(x, wa_ref[...], preferred_element_type=jnp.float32)
  yb_ref[...] = jnp.dot(x, wb_ref[...], preferred_element_type=jnp.float32)
  yc_ref[...] = jnp.dot(x, wc_ref[...], preferred_element_type=jnp.float32)


_project = pl.pallas_call(
    _project_body,
    grid=(N // _PROJ_BLK,),
    in_specs=[
        pl.BlockSpec((_PROJ_BLK, D), lambda i: (i, 0)),
        pl.BlockSpec((D, D), lambda i: (0, 0)),
        pl.BlockSpec((D, D), lambda i: (0, 0)),
        pl.BlockSpec((D, D), lambda i: (0, 0)),
    ],
    out_specs=[pl.BlockSpec((_PROJ_BLK, D), lambda i: (i, 0))] * 3,
    out_shape=[jax.ShapeDtypeStruct((N, D), jnp.float32)] * 3,
)


_CHUNK = 80
_BLK = 2000
_NCHUNK_B = _BLK // _CHUNK
_NBLK = EPW // _BLK
_RPT = 632
_RPT_LAST = N - 15 * _RPT


def _row_chunks(total):
  out, off = [], 0
  while off < total:
    n = min(_CHUNK, total - off)
    out.append((off, n))
    off += n
  return out


def _msgpass_body(y0, y1, y2, e0s, e0d, e1s, e1d, e2s, e2d, w0, w1, w2,
                  normsf, out_hbm, a_hbm,
                  nsrc_v, ndst_v, src_f, dst_f, ew_v, c_v, rows0, rows1,
                  sem0, sem1, sem2, sem3,
                  acc_sh, a0_sh, a1_sh, a2_sh):
  c = lax.axis_index("c")
  s = lax.axis_index("s")
  a_shs = [a0_sh, a1_sh, a2_sh]

  def init_zrows(i, _):
    for j in range(D // 16):
      rows0[i, pl.ds(j * 16, 16)] = jnp.full((16,), 0.0, jnp.float32)
    return 0
  lax.fori_loop(0, _CHUNK, init_zrows, 0)

  def init_znsrc(i, _):
    nsrc_v[pl.ds(i * 16, 16)] = jnp.full((16,), 0.0, jnp.float32)
    return 0
  lax.fori_loop(0, N // 16, init_znsrc, 0)

  start = s * _RPT
  @pl.when(s < 15)
  def _():
    for off, n in _row_chunks(_RPT):
      pltpu.sync_copy(rows0.at[pl.ds(0, n), :],
                      acc_sh.at[pl.ds(start + off, n), :])
  @pl.when(s == 15)
  def _():
    for off, n in _row_chunks(_RPT_LAST):
      pltpu.sync_copy(rows0.at[pl.ds(0, n), :],
                      acc_sh.at[pl.ds(15 * _RPT + off, n), :])
  for r in range(3):
    @pl.when(s == r)
    def _():
      pltpu.sync_copy(nsrc_v, a_shs[r])
  plsc.subcore_barrier()

  w = c * NS + s
  base_w = w * EPW

  def _scale(rows_ref, kc):
    @plsc.parallel_loop(0, _CHUNK, 1, unroll=4)
    def _(e):
      cb = plsc.load_gather(
          c_v, [jnp.full((16,), 0, jnp.int32) + (kc * _CHUNK + e)])
      for j in range(D // 16):
        rows_ref[e, pl.ds(j * 16, 16)] = rows_ref[e, pl.ds(j * 16, 16)] * cb

  for r, (y, es, ed, ew) in enumerate(
      ((y0, e0s, e0d, w0), (y1, e1s, e1d, w1), (y2, e2s, e2d, w2))):
    pltpu.sync_copy(normsf.at[pl.ds((2 * r) * N, N)], nsrc_v)
    pltpu.sync_copy(normsf.at[pl.ds((2 * r + 1) * N, N)], ndst_v)

    def block(bi, _):
      base = base_w + bi * _BLK
      ld0 = pltpu.async_copy(es.at[pl.ds(base, _BLK)], src_f, sem2)
      ld1 = pltpu.async_copy(ed.at[pl.ds(base, _BLK)], dst_f, sem2)
      ld2 = pltpu.async_copy(ew.at[pl.ds(base, _BLK)], ew_v, sem2)
      ld0.wait(); ld1.wait(); ld2.wait()

      def cgrp(g, _):
        s16 = src_f[pl.ds(g * 16, 16)]
        d16 = dst_f[pl.ds(g * 16, 16)]
        ns16 = plsc.load_gather(nsrc_v, [s16])
        nd16 = plsc.load_gather(ndst_v, [d16])
        c_v[pl.ds(g * 16, 16)] = ns16 * nd16 * ew_v[pl.ds(g * 16, 16)]
        return 0
      lax.fori_loop(0, _BLK // 16, cgrp, 0)

      a_scat = pltpu.async_copy(c_v, a_shs[r].at[src_f], sem3, add=True)

      pltpu.async_copy(y.at[src_f.at[pl.ds(0, _CHUNK)]], rows0, sem0)

      def two(k2, _):
        k = 2 * k2
        pltpu.async_copy(
            y.at[src_f.at[pl.ds((k + 1) * _CHUNK, _CHUNK)]], rows1, sem1)
        pltpu.make_async_copy(
            y.at[src_f.at[pl.ds(0, _CHUNK)]], rows0, sem0).wait()
        _scale(rows0, k)
        pltpu.sync_copy(rows0, acc_sh.at[dst_f.at[pl.ds(k * _CHUNK, _CHUNK)]],
                        add=True)
        pltpu.async_copy(
            y.at[src_f.at[pl.ds((k + 2) * _CHUNK, _CHUNK)]], rows0, sem0)
        pltpu.make_async_copy(
            y.at[src_f.at[pl.ds(0, _CHUNK)]], rows1, sem1).wait()
        _scale(rows1, k + 1)
        pltpu.sync_copy(
            rows1, acc_sh.at[dst_f.at[pl.ds((k + 1) * _CHUNK, _CHUNK)]],
            add=True)
        return 0
      lax.fori_loop(0, (_NCHUNK_B - 1) // 2, two, 0)

      last = _NCHUNK_B - 1
      pltpu.make_async_copy(
          y.at[src_f.at[pl.ds(0, _CHUNK)]], rows0, sem0).wait()
      _scale(rows0, last)
      pltpu.sync_copy(rows0, acc_sh.at[dst_f.at[pl.ds(last * _CHUNK, _CHUNK)]],
                      add=True)
      a_scat.wait()
      return 0
    lax.fori_loop(0, _NBLK, block, 0)

  plsc.subcore_barrier()

  def _flush(row0, nrows):
    pltpu.sync_copy(acc_sh.at[pl.ds(row0, nrows), :],
                    rows0.at[pl.ds(0, nrows), :])
    pltpu.sync_copy(rows0.at[pl.ds(0, nrows), :],
                    out_hbm.at[pl.ds(c * N + row0, nrows), :])

  @pl.when(s < 15)
  def _():
    for off, n in _row_chunks(_RPT):
      _flush(start + off, n)
  @pl.when(s == 15)
  def _():
    for off, n in _row_chunks(_RPT_LAST):
      _flush(15 * _RPT + off, n)
  for r in range(3):
    @pl.when(s == r)
    def _():
      pltpu.sync_copy(a_shs[r], nsrc_v)
      pltpu.sync_copy(nsrc_v, a_hbm.at[pl.ds((c * 3 + r) * N, N)])


_msgpass = pl.kernel(
    _msgpass_body,
    out_type=[
        jax.ShapeDtypeStruct((NC * N, D), jnp.float32),
        jax.ShapeDtypeStruct((NC * 3 * N,), jnp.float32),
    ],
    mesh=_SC_MESH,
    compiler_params=pltpu.CompilerParams(needs_layout_passes=False),
    scratch_types=[
        pltpu.VMEM((N,), jnp.float32),
        pltpu.VMEM((N,), jnp.float32),
        pltpu.VMEM((_BLK,), jnp.int32),
        pltpu.VMEM((_BLK,), jnp.int32),
        pltpu.VMEM((_BLK,), jnp.float32),
        pltpu.VMEM((_BLK,), jnp.float32),
        pltpu.VMEM((_CHUNK, D), jnp.float32),
        pltpu.VMEM((_CHUNK, D), jnp.float32),
        pltpu.SemaphoreType.DMA,
        pltpu.SemaphoreType.DMA,
        pltpu.SemaphoreType.DMA,
        pltpu.SemaphoreType.DMA,
        pltpu.VMEM_SHARED((N, D), jnp.float32),
        pltpu.VMEM_SHARED((N,), jnp.float32),
        pltpu.VMEM_SHARED((N,), jnp.float32),
        pltpu.VMEM_SHARED((N,), jnp.float32),
    ],
)


def _final_body(out1_ref, a_ref, wa_ref, wb_ref, wc_ref, hg_ref):
  h1 = jax.nn.relu(out1_ref[0] + out1_ref[1])
  a = a_ref[0] + a_ref[1]
  sums = jnp.dot(a, h1, preferred_element_type=jnp.float32)
  hg = (jnp.dot(sums[0:1], wa_ref[...], preferred_element_type=jnp.float32)
        + jnp.dot(sums[1:2], wb_ref[...], preferred_element_type=jnp.float32)
        + jnp.dot(sums[2:3], wc_ref[...], preferred_element_type=jnp.float32))
  hg_ref[...] = hg * (1.0 / N)


_final = pl.pallas_call(
    _final_body,
    out_shape=jax.ShapeDtypeStruct((1, D), jnp.float32),
)


@jax.jit
def kernel(x, edge_index_r0, edge_index_r1, edge_index_r2,
           w_r0, w_r1, w_r2,
           W0_r0, W0_r1, W0_r2, W1_r0, W1_r1, W1_r2):
  es = [edge_index_r0.astype(jnp.int32), edge_index_r1.astype(jnp.int32),
        edge_index_r2.astype(jnp.int32)]
  e0s, e0d = es[0][0], es[0][1]
  e1s, e1d = es[1][0], es[1][1]
  e2s, e2d = es[2][0], es[2][1]
  degf = _degrees(e0s, e0d, e1s, e1d, e2s, e2d)
  norms = _norms(degf.reshape(NC, 6, N))
  y0, y1, y2 = _project(x, W0_r0, W0_r1, W0_r2)
  out1f, af = _msgpass(y0, y1, y2, e0s, e0d, e1s, e1d, e2s, e2d,
                       w_r0, w_r1, w_r2, norms.reshape(-1))
  return _final(out1f.reshape(NC, N, D), af.reshape(NC, 3, N),
                W1_r0, W1_r1, W1_r2)
